# baseline (speedup 1.0000x reference)
import functools

import jax
import jax.numpy as jnp
from jax import lax
from jax.experimental import pallas as pl
from jax.experimental.pallas import tpu as pltpu
from jax.experimental.pallas import tpu_sc as plsc

N = 10000
E = 320000
F = 128
G = 64
C = 10

NC = 2
NS = 16
NW = NC * NS
CHUNK = 64
EDGES_PER_W = 10240
E_PAD = NW * EDGES_PER_W
NCHUNK = EDGES_PER_W // CHUNK
NPHASE = 2
PCHUNK = NCHUNK // NPHASE
N_PAD = 10240
ROWS_PER_TILE = N_PAD // NS

_mesh = plsc.VectorSubcoreMesh(core_axis_name="c", subcore_axis_name="s")


NBUF = 2


@functools.partial(
    pl.kernel,
    out_type=jax.ShapeDtypeStruct((NC, N_PAD, F), jnp.float32),
    mesh=_mesh,
    scratch_types=[
        pltpu.VMEM((2, PCHUNK, CHUNK), jnp.int32),
        pltpu.VMEM((CHUNK, F), jnp.float32),
        pltpu.VMEM_SHARED((N_PAD, F), jnp.float32),
        pltpu.SemaphoreType.DMA,
    ],
)
def _sc_aggregate(h_hbm, ei_hbm, zeros_hbm, out_hbm, idx_v, rows_v, acc_sh, s0):
    cid = lax.axis_index("c")
    sid = lax.axis_index("s")
    wid = sid * NC + cid

    pltpu.sync_copy(zeros_hbm, acc_sh.at[pl.ds(sid * ROWS_PER_TILE, ROWS_PER_TILE)])
    plsc.subcore_barrier()

    for p in range(NPHASE):
        pltpu.sync_copy(ei_hbm.at[0, wid, pl.ds(p * PCHUNK, PCHUNK)], idx_v.at[0])
        pltpu.sync_copy(ei_hbm.at[1, wid, pl.ds(p * PCHUNK, PCHUNK)], idx_v.at[1])

        def body(c, carry):
            pltpu.async_copy(h_hbm.at[idx_v.at[0, c]], rows_v, s0).wait()
            pltpu.sync_copy(rows_v, acc_sh.at[idx_v.at[1, c]], add=True)
            return carry

        lax.fori_loop(0, PCHUNK, body, 0)

    plsc.subcore_barrier()

    pltpu.sync_copy(acc_sh.at[pl.ds(sid * ROWS_PER_TILE, ROWS_PER_TILE)],
                    out_hbm.at[cid, pl.ds(sid * ROWS_PER_TILE, ROWS_PER_TILE)])


def _mlp_body(h_ref, a0_ref, a1_ref, wa_ref, ba_ref, wb_ref, bb_ref, o_ref):
    z = h_ref[...] + a0_ref[...] + a1_ref[...]
    z = jnp.dot(z, wa_ref[...], preferred_element_type=jnp.float32) + ba_ref[...]
    z = jnp.maximum(z, 0.0)
    z = jnp.dot(z, wb_ref[...], preferred_element_type=jnp.float32) + bb_ref[...]
    o_ref[...] = jnp.maximum(z, 0.0)


_ROWS_BLK = 1000


def _tc_mlp(h, a0, a1, wa, ba, wb, bb):
    grid = (N // _ROWS_BLK,)
    return pl.pallas_call(
        _mlp_body,
        grid=grid,
        in_specs=[
            pl.BlockSpec((_ROWS_BLK, F), lambda i: (i, 0)),
            pl.BlockSpec((_ROWS_BLK, F), lambda i: (i, 0)),
            pl.BlockSpec((_ROWS_BLK, F), lambda i: (i, 0)),
            pl.BlockSpec((F, F), lambda i: (0, 0)),
            pl.BlockSpec((1, F), lambda i: (0, 0)),
            pl.BlockSpec((F, F), lambda i: (0, 0)),
            pl.BlockSpec((1, F), lambda i: (0, 0)),
        ],
        out_specs=pl.BlockSpec((_ROWS_BLK, F), lambda i: (i, 0)),
        out_shape=jax.ShapeDtypeStruct((N, F), jnp.float32),
    )(h, a0, a1, wa, ba.reshape(1, F), wb, bb.reshape(1, F))


def _pool_body(h_ref, batch_ref, wfc_ref, bfc_ref, o_ref):
    gids = lax.broadcasted_iota(jnp.int32, (G, N), 0)
    onehot = (gids == batch_ref[...]).astype(jnp.float32)
    pooled = jnp.dot(onehot, h_ref[...], preferred_element_type=jnp.float32)
    logits = jnp.dot(pooled, wfc_ref[...], preferred_element_type=jnp.float32)
    logits = logits + bfc_ref[...]
    m = jnp.max(logits, axis=-1, keepdims=True)
    shifted = logits - m
    lse = jnp.log(jnp.sum(jnp.exp(shifted), axis=-1, keepdims=True))
    o_ref[...] = shifted - lse


def _tc_pool(h, batch, wfc, bfc):
    return pl.pallas_call(
        _pool_body,
        out_shape=jax.ShapeDtypeStruct((G, C), jnp.float32),
    )(h, batch.reshape(1, N), wfc, bfc.reshape(1, C))


def kernel(x, edge_index, batch, W1a, b1a, W1b, b1b, W2a, b2a, W2b, b2b, Wfc, bfc):
    pad = jnp.tile(jnp.array([[0], [N]], jnp.int32), (1, E_PAD - E))
    ei = jnp.concatenate([edge_index, pad], axis=1).reshape(2, NW, NCHUNK, CHUNK)
    zeros = jnp.zeros((ROWS_PER_TILE, F), jnp.float32)

    agg1 = _sc_aggregate(x, ei, zeros)
    h1 = _tc_mlp(x, agg1[0, :N], agg1[1, :N], W1a, b1a, W1b, b1b)
    agg2 = _sc_aggregate(h1, ei, zeros)
    h2 = _tc_mlp(h1, agg2[0, :N], agg2[1, :N], W2a, b2a, W2b, b2b)
    return _tc_pool(h2, batch, Wfc, bfc)


# speedup vs baseline: 2.4921x; 2.4921x over previous
import functools

import jax
import jax.numpy as jnp
from jax import lax
from jax.experimental import pallas as pl
from jax.experimental.pallas import tpu as pltpu
from jax.experimental.pallas import tpu_sc as plsc

N = 10000
E = 320000
F = 128
G = 64
C = 10

NC = 2
NS = 16
NW = NC * NS
CHUNK = 80
EDGES_PER_W = 10000
E_PAD = NW * EDGES_PER_W
NCHUNK = EDGES_PER_W // CHUNK
N_PAD = 10240
ROWS_PER_TILE = N_PAD // NS

_mesh = plsc.VectorSubcoreMesh(core_axis_name="c", subcore_axis_name="s")


NBUF = 2


@functools.partial(
    pl.kernel,
    out_type=jax.ShapeDtypeStruct((NC, N_PAD, F), jnp.float32),
    mesh=_mesh,
    scratch_types=[
        pltpu.VMEM((2, NCHUNK, CHUNK), jnp.int32),
        pltpu.VMEM((CHUNK, F), jnp.float32),
        pltpu.VMEM_SHARED((N_PAD, F), jnp.float32),
        pltpu.SemaphoreType.DMA,
    ],
)
def _sc_aggregate(h_hbm, ei_hbm, zeros_hbm, out_hbm, idx_v, rows_v, acc_sh, s0):
    cid = lax.axis_index("c")
    sid = lax.axis_index("s")
    wid = sid * NC + cid

    pltpu.sync_copy(zeros_hbm, acc_sh.at[pl.ds(sid * ROWS_PER_TILE, ROWS_PER_TILE)])
    pltpu.sync_copy(ei_hbm.at[0, wid], idx_v.at[0])
    pltpu.sync_copy(ei_hbm.at[1, wid], idx_v.at[1])
    plsc.subcore_barrier()

    def body(c, carry):
        pltpu.async_copy(h_hbm.at[idx_v.at[0, c]], rows_v, s0).wait()
        pltpu.sync_copy(rows_v, acc_sh.at[idx_v.at[1, c]], add=True)
        return carry

    lax.fori_loop(0, NCHUNK, body, 0)
    plsc.subcore_barrier()

    pltpu.sync_copy(acc_sh.at[pl.ds(sid * ROWS_PER_TILE, ROWS_PER_TILE)],
                    out_hbm.at[cid, pl.ds(sid * ROWS_PER_TILE, ROWS_PER_TILE)])


_ROWS_BLK = 1024
_NBLK = N_PAD // _ROWS_BLK


def _mlp_body(h_ref, a0_ref, a1_ref, wa_ref, ba_ref, wb_ref, bb_ref, o_ref):
    z = h_ref[...] + a0_ref[...] + a1_ref[...]
    z = jnp.dot(z, wa_ref[...], preferred_element_type=jnp.float32) + ba_ref[...]
    z = jnp.maximum(z, 0.0)
    z = jnp.dot(z, wb_ref[...], preferred_element_type=jnp.float32) + bb_ref[...]
    o_ref[...] = jnp.maximum(z, 0.0)


def _tc_mlp(h, a0, a1, wa, ba, wb, bb):
    return pl.pallas_call(
        _mlp_body,
        grid=(_NBLK,),
        in_specs=[
            pl.BlockSpec((_ROWS_BLK, F), lambda i: (i, 0)),
            pl.BlockSpec((_ROWS_BLK, F), lambda i: (i, 0)),
            pl.BlockSpec((_ROWS_BLK, F), lambda i: (i, 0)),
            pl.BlockSpec((F, F), lambda i: (0, 0)),
            pl.BlockSpec((1, F), lambda i: (0, 0)),
            pl.BlockSpec((F, F), lambda i: (0, 0)),
            pl.BlockSpec((1, F), lambda i: (0, 0)),
        ],
        out_specs=pl.BlockSpec((_ROWS_BLK, F), lambda i: (i, 0)),
        out_shape=jax.ShapeDtypeStruct((N_PAD, F), jnp.float32),
    )(h, a0, a1, wa, ba.reshape(1, F), wb, bb.reshape(1, F))


def _mlp_pool_body(h_ref, a0_ref, a1_ref, wa_ref, ba_ref, wb_ref, bb_ref,
                   batch_ref, wfc_ref, bfc_ref, o_ref, pooled_acc):
    i = pl.program_id(0)
    z = h_ref[...] + a0_ref[...] + a1_ref[...]
    z = jnp.dot(z, wa_ref[...], preferred_element_type=jnp.float32) + ba_ref[...]
    z = jnp.maximum(z, 0.0)
    z = jnp.dot(z, wb_ref[...], preferred_element_type=jnp.float32) + bb_ref[...]
    h2 = jnp.maximum(z, 0.0)
    gids = lax.broadcasted_iota(jnp.int32, (G, _ROWS_BLK), 0)
    onehot = (gids == batch_ref[...]).astype(jnp.float32)
    part = jnp.dot(onehot, h2, preferred_element_type=jnp.float32)

    @pl.when(i == 0)
    def _():
        pooled_acc[...] = part

    @pl.when(i > 0)
    def _():
        pooled_acc[...] += part

    @pl.when(i == _NBLK - 1)
    def _():
        logits = jnp.dot(pooled_acc[...], wfc_ref[...],
                         preferred_element_type=jnp.float32) + bfc_ref[...]
        m = jnp.max(logits, axis=-1, keepdims=True)
        shifted = logits - m
        lse = jnp.log(jnp.sum(jnp.exp(shifted), axis=-1, keepdims=True))
        o_ref[...] = shifted - lse


def _tc_mlp_pool(h, a0, a1, wa, ba, wb, bb, batch_pad, wfc, bfc):
    return pl.pallas_call(
        _mlp_pool_body,
        grid=(_NBLK,),
        in_specs=[
            pl.BlockSpec((_ROWS_BLK, F), lambda i: (i, 0)),
            pl.BlockSpec((_ROWS_BLK, F), lambda i: (i, 0)),
            pl.BlockSpec((_ROWS_BLK, F), lambda i: (i, 0)),
            pl.BlockSpec((F, F), lambda i: (0, 0)),
            pl.BlockSpec((1, F), lambda i: (0, 0)),
            pl.BlockSpec((F, F), lambda i: (0, 0)),
            pl.BlockSpec((1, F), lambda i: (0, 0)),
            pl.BlockSpec((1, _ROWS_BLK), lambda i: (0, i)),
            pl.BlockSpec((F, C), lambda i: (0, 0)),
            pl.BlockSpec((1, C), lambda i: (0, 0)),
        ],
        out_specs=pl.BlockSpec((G, C), lambda i: (0, 0)),
        out_shape=jax.ShapeDtypeStruct((G, C), jnp.float32),
        scratch_shapes=[pltpu.VMEM((G, F), jnp.float32)],
    )(h, a0, a1, wa, ba.reshape(1, F), wb, bb.reshape(1, F),
      batch_pad, wfc, bfc.reshape(1, C))


def kernel(x, edge_index, batch, W1a, b1a, W1b, b1b, W2a, b2a, W2b, b2b, Wfc, bfc):
    ei = edge_index.reshape(2, NW, NCHUNK, CHUNK)
    zeros = jnp.zeros((ROWS_PER_TILE, F), jnp.float32)
    x_pad = jnp.pad(x, ((0, N_PAD - N), (0, 0)))
    batch_pad = jnp.pad(batch, (0, N_PAD - N), constant_values=G).reshape(1, N_PAD)

    agg1 = _sc_aggregate(x_pad, ei, zeros)
    h1 = _tc_mlp(x_pad, agg1[0], agg1[1], W1a, b1a, W1b, b1b)
    agg2 = _sc_aggregate(h1, ei, zeros)
    return _tc_mlp_pool(h1, agg2[0], agg2[1], W2a, b2a, W2b, b2b,
                        batch_pad, Wfc, bfc)
